# baseline (speedup 1.0000x reference)
import functools

import jax
import jax.numpy as jnp
from jax import lax
from jax.experimental import pallas as pl
from jax.experimental.pallas import tpu as pltpu
from jax.experimental.pallas import tpu_sc as plsc

N = 10000
E = 320000
H = 128

NC = 2
NS = 16
NW = NC * NS
EP = 327680
EPT = EP // NW
K = 128
ST = 4
CPS = 20
NP = 10112
RPT = NP // NS

F32 = jnp.float32


def _mesh():
    return plsc.VectorSubcoreMesh(
        core_axis_name="c", subcore_axis_name="s",
        num_cores=NC, num_subcores=NS)


def _sc_deg(dsts):
    out_t = (jax.ShapeDtypeStruct((NC, NP, 16), F32),
             jax.ShapeDtypeStruct((NC, NP, 16), F32))

    @functools.partial(
        pl.kernel,
        out_type=out_t,
        mesh=_mesh(),
        scratch_types=[
            pltpu.VMEM((CPS, K), jnp.int32),
            pltpu.VMEM((K, 16), F32),
            pltpu.VMEM_SHARED((NP, 16), F32),
        ],
    )
    def k(dsts_hbm, oi_hbm, os_hbm, idx_v, ones_v, acc):
        c = lax.axis_index("c")
        s = lax.axis_index("s")
        wid = c * NS + s

        def _fill(val):
            def _f(i, carry):
                ones_v[i, :] = jnp.full((16,), val, F32)
                return carry
            lax.fori_loop(0, K, _f, 0)

        for g, out_hbm in enumerate((oi_hbm, os_hbm)):
            _fill(0.0)
            for j in range(RPT // K):
                pltpu.sync_copy(ones_v, acc.at[pl.ds(s * RPT + j * K, K)])
            pltpu.sync_copy(ones_v.at[pl.ds(0, RPT % K)],
                            acc.at[pl.ds(s * RPT + (RPT // K) * K, RPT % K)])
            plsc.subcore_barrier()
            _fill(1.0)

            for b in range(ST):
                pltpu.sync_copy(dsts_hbm.at[g, wid, b], idx_v)

                def _step(r, carry):
                    pltpu.sync_copy(ones_v, acc.at[idx_v.at[r]], add=True)
                    return carry
                lax.fori_loop(0, CPS, _step, 0)
            plsc.subcore_barrier()

            pltpu.sync_copy(acc.at[pl.ds(s * RPT, RPT)],
                            out_hbm.at[c, pl.ds(s * RPT, RPT)])

    return k(dsts)


def _sc_scatter(xs, src, dst):
    @functools.partial(
        pl.kernel,
        out_type=jax.ShapeDtypeStruct((NC, NP, H), F32),
        mesh=_mesh(),
        scratch_types=[
            pltpu.VMEM((CPS, K), jnp.int32),
            pltpu.VMEM((CPS, K), jnp.int32),
            pltpu.VMEM((K, H), F32),
            pltpu.VMEM((K, H), F32),
            pltpu.VMEM_SHARED((NP, H), F32),
            pltpu.SemaphoreType.DMA,
            pltpu.SemaphoreType.DMA,
        ],
    )
    def k(xs_hbm, src_hbm, dst_hbm, out_hbm,
          src_v, dst_v, rows0, rows1, acc, sem0, sem1):
        c = lax.axis_index("c")
        s = lax.axis_index("s")
        wid = c * NS + s
        base = s * RPT

        def _zrow(i, carry):
            for j in range(H // 16):
                rows0[i, pl.ds(j * 16, 16)] = jnp.zeros((16,), F32)
                rows1[i, pl.ds(j * 16, 16)] = jnp.zeros((16,), F32)
            return carry
        lax.fori_loop(0, K, _zrow, 0)
        for j in range(RPT // (2 * K)):
            pltpu.sync_copy(rows0, acc.at[pl.ds(base + j * 2 * K, K)])
            pltpu.sync_copy(rows1, acc.at[pl.ds(base + j * 2 * K + K, K)])
        pltpu.sync_copy(rows0.at[pl.ds(0, RPT % (2 * K))],
                        acc.at[pl.ds(base + (RPT // (2 * K)) * 2 * K,
                                     RPT % (2 * K))])
        plsc.subcore_barrier()

        for b in range(ST):
            pltpu.sync_copy(src_hbm.at[wid, b], src_v)
            pltpu.sync_copy(dst_hbm.at[wid, b], dst_v)

            def _pair(i, carry):
                r = 2 * i
                d0 = pltpu.async_copy(xs_hbm.at[src_v.at[r]], rows0, sem0)
                d1 = pltpu.async_copy(xs_hbm.at[src_v.at[r + 1]], rows1, sem1)
                d0.wait()
                pltpu.sync_copy(rows0, acc.at[dst_v.at[r]], add=True)
                d1.wait()
                pltpu.sync_copy(rows1, acc.at[dst_v.at[r + 1]], add=True)
                return carry
            lax.fori_loop(0, CPS // 2, _pair, 0)
        plsc.subcore_barrier()

        pltpu.sync_copy(acc.at[pl.ds(s * RPT, RPT)],
                        out_hbm.at[c, pl.ds(s * RPT, RPT)])

    return k(xs, src, dst)


_R = 1000
_GRID = (N // _R,)


def _dinv_of(dp):
    return lax.rsqrt(1.0 + dp[0, :, 0] + dp[1, :, 0])[:, None]


def _row_spec():
    return pl.BlockSpec((_R, H), lambda i: (i, 0))


def _w_spec():
    return pl.BlockSpec((H, H), lambda i: (0, 0))


def _deg_spec():
    return pl.BlockSpec((NC, _R, 16), lambda i: (0, i, 0))


def _part_spec():
    return pl.BlockSpec((NC, _R, H), lambda i: (0, i, 0))


def _b_spec():
    return pl.BlockSpec((1, H), lambda i: (0, 0))


def _prep_body(x_ref, wt_ref, dp_ref, o_ref):
    dinv = _dinv_of(dp_ref[...])
    o_ref[...] = dinv * jnp.dot(x_ref[...], wt_ref[...],
                                preferred_element_type=F32)


def _tc_prep(x, wt, degp):
    return pl.pallas_call(
        _prep_body,
        grid=_GRID,
        in_specs=[_row_spec(), _w_spec(), _deg_spec()],
        out_specs=_row_spec(),
        out_shape=jax.ShapeDtypeStruct((N, H), F32),
    )(x, wt, degp)


def _mid_body(p_ref, xs_ref, dp_ref, b_ref, wt_ref, o_ref):
    dinv = _dinv_of(dp_ref[...])
    p = p_ref[...]
    h = jnp.maximum(dinv * (p[0] + p[1] + xs_ref[...]) + b_ref[...], 0.0)
    o_ref[...] = dinv * jnp.dot(h, wt_ref[...], preferred_element_type=F32)


def _tc_mid(parts, xs, degp, b, wt):
    return pl.pallas_call(
        _mid_body,
        grid=_GRID,
        in_specs=[_part_spec(), _row_spec(), _deg_spec(), _b_spec(),
                  _w_spec()],
        out_specs=_row_spec(),
        out_shape=jax.ShapeDtypeStruct((N, H), F32),
    )(parts, xs, degp, b, wt)


def _fin_body(pi_ref, xsi_ref, dpi_ref, bi_ref,
              ps_ref, xss_ref, dps_ref, bs_ref,
              fwi_ref, fws_ref, fb_ref, o_ref):
    dinv_i = _dinv_of(dpi_ref[...])
    pi = pi_ref[...]
    h_i = jnp.maximum(
        dinv_i * (pi[0] + pi[1] + xsi_ref[...]) + bi_ref[...], 0.0)
    dinv_s = _dinv_of(dps_ref[...])
    ps = ps_ref[...]
    h_s = jnp.maximum(
        dinv_s * (ps[0] + ps[1] + xss_ref[...]) + bs_ref[...], 0.0)
    o_ref[...] = (jnp.dot(h_i, fwi_ref[...], preferred_element_type=F32)
                  + jnp.dot(h_s, fws_ref[...], preferred_element_type=F32)
                  + fb_ref[...])


def _tc_fin(pi, xsi, dpi, bi, ps, xss, dps, bs, fwi, fws, fb):
    return pl.pallas_call(
        _fin_body,
        grid=_GRID,
        in_specs=[_part_spec(), _row_spec(), _deg_spec(), _b_spec(),
                  _part_spec(), _row_spec(), _deg_spec(), _b_spec(),
                  _w_spec(), _w_spec(), _b_spec()],
        out_specs=_row_spec(),
        out_shape=jax.ShapeDtypeStruct((N, H), F32),
    )(pi, xsi, dpi, bi, ps, xss, dps, bs, fwi, fws, fb)


def kernel(interaction_x, interaction_edge_index,
           similarity_x, similarity_edge_index,
           W_ic1, b_ic1, W_ic2, b_ic2,
           W_sc1, b_sc1, W_sc2, b_sc2, fc_W, fc_b):
    zpad = jnp.zeros((EP - E,), jnp.int32)
    npad = jnp.full((EP - E,), N, jnp.int32)
    src_i = jnp.concatenate(
        [interaction_edge_index[0], zpad]).reshape(NW, ST, CPS, K)
    dst_i = jnp.concatenate(
        [interaction_edge_index[1], npad]).reshape(NW, ST, CPS, K)
    src_s = jnp.concatenate(
        [similarity_edge_index[0], zpad]).reshape(NW, ST, CPS, K)
    dst_s = jnp.concatenate(
        [similarity_edge_index[1], npad]).reshape(NW, ST, CPS, K)
    dsts = jnp.stack([dst_i, dst_s])

    degp_i, degp_s = _sc_deg(dsts)

    b1_i = b_ic1.reshape(1, H)
    b2_i = b_ic2.reshape(1, H)
    b1_s = b_sc1.reshape(1, H)
    b2_s = b_sc2.reshape(1, H)
    fwt = fc_W.T
    fwt_i = fwt[:H]
    fwt_s = fwt[H:]
    fb = fc_b.reshape(1, H)

    xs1_i = _tc_prep(interaction_x, W_ic1.T, degp_i)
    xs1_s = _tc_prep(similarity_x, W_sc1.T, degp_s)
    p1_i = _sc_scatter(xs1_i, src_i, dst_i)
    p1_s = _sc_scatter(xs1_s, src_s, dst_s)
    xs2_i = _tc_mid(p1_i, xs1_i, degp_i, b1_i, W_ic2.T)
    xs2_s = _tc_mid(p1_s, xs1_s, degp_s, b1_s, W_sc2.T)
    p2_i = _sc_scatter(xs2_i, src_i, dst_i)
    p2_s = _sc_scatter(xs2_s, src_s, dst_s)
    return _tc_fin(p2_i, xs2_i, degp_i, b2_i,
                   p2_s, xs2_s, degp_s, b2_s,
                   fwt_i, fwt_s, fb)


# speedup vs baseline: 1.0182x; 1.0182x over previous
import functools

import jax
import jax.numpy as jnp
from jax import lax
from jax.experimental import pallas as pl
from jax.experimental.pallas import tpu as pltpu
from jax.experimental.pallas import tpu_sc as plsc

N = 10000
E = 320000
H = 128

NC = 2
NS = 16
NW = NC * NS
EP = 327680
EPT = EP // NW
K = 128
ST = 4
CPS = 20
NP = 10112
RPT = NP // NS

F32 = jnp.float32


def _mesh():
    return plsc.VectorSubcoreMesh(
        core_axis_name="c", subcore_axis_name="s",
        num_cores=NC, num_subcores=NS)


def _sc_deg(dsts):
    out_t = (jax.ShapeDtypeStruct((NC, NP, 16), F32),
             jax.ShapeDtypeStruct((NC, NP, 16), F32))

    @functools.partial(
        pl.kernel,
        out_type=out_t,
        mesh=_mesh(),
        scratch_types=[
            pltpu.VMEM((CPS, K), jnp.int32),
            pltpu.VMEM((K, 16), F32),
            pltpu.VMEM_SHARED((NP, 16), F32),
        ],
    )
    def k(dsts_hbm, oi_hbm, os_hbm, idx_v, ones_v, acc):
        c = lax.axis_index("c")
        s = lax.axis_index("s")
        wid = c * NS + s

        def _fill(val):
            def _f(i, carry):
                ones_v[i, :] = jnp.full((16,), val, F32)
                return carry
            lax.fori_loop(0, K, _f, 0)

        for g, out_hbm in enumerate((oi_hbm, os_hbm)):
            _fill(0.0)
            for j in range(RPT // K):
                pltpu.sync_copy(ones_v, acc.at[pl.ds(s * RPT + j * K, K)])
            pltpu.sync_copy(ones_v.at[pl.ds(0, RPT % K)],
                            acc.at[pl.ds(s * RPT + (RPT // K) * K, RPT % K)])
            plsc.subcore_barrier()
            _fill(1.0)

            for b in range(ST):
                pltpu.sync_copy(dsts_hbm.at[g, wid, b], idx_v)

                def _step(r, carry):
                    pltpu.sync_copy(ones_v, acc.at[idx_v.at[r]], add=True)
                    return carry
                lax.fori_loop(0, CPS, _step, 0)
            plsc.subcore_barrier()

            pltpu.sync_copy(acc.at[pl.ds(s * RPT, RPT)],
                            out_hbm.at[c, pl.ds(s * RPT, RPT)])

    return k(dsts)


def _sc_scatter(xs, src, dst):
    @functools.partial(
        pl.kernel,
        out_type=jax.ShapeDtypeStruct((NC, NP, H), F32),
        mesh=_mesh(),
        scratch_types=[
            pltpu.VMEM((CPS, K), jnp.int32),
            pltpu.VMEM((CPS, K), jnp.int32),
            pltpu.VMEM((K, H), F32),
            pltpu.VMEM((K, H), F32),
            pltpu.VMEM_SHARED((NP, H), F32),
            pltpu.SemaphoreType.DMA,
            pltpu.SemaphoreType.DMA,
        ],
    )
    def k(xs_hbm, src_hbm, dst_hbm, out_hbm,
          src_v, dst_v, rows0, rows1, acc, sem0, sem1):
        c = lax.axis_index("c")
        s = lax.axis_index("s")
        wid = c * NS + s
        base = s * RPT

        def _zrow(i, carry):
            for j in range(H // 16):
                rows0[i, pl.ds(j * 16, 16)] = jnp.zeros((16,), F32)
                rows1[i, pl.ds(j * 16, 16)] = jnp.zeros((16,), F32)
            return carry
        lax.fori_loop(0, K, _zrow, 0)
        for j in range(RPT // (2 * K)):
            pltpu.sync_copy(rows0, acc.at[pl.ds(base + j * 2 * K, K)])
            pltpu.sync_copy(rows1, acc.at[pl.ds(base + j * 2 * K + K, K)])
        pltpu.sync_copy(rows0.at[pl.ds(0, RPT % (2 * K))],
                        acc.at[pl.ds(base + (RPT // (2 * K)) * 2 * K,
                                     RPT % (2 * K))])
        plsc.subcore_barrier()

        for b in range(ST):
            pltpu.sync_copy(src_hbm.at[wid, b], src_v)
            pltpu.sync_copy(dst_hbm.at[wid, b], dst_v)

            def _pair(i, carry):
                r = 2 * i
                d0 = pltpu.async_copy(xs_hbm.at[src_v.at[r]], rows0, sem0)
                d1 = pltpu.async_copy(xs_hbm.at[src_v.at[r + 1]], rows1, sem1)
                d0.wait()
                pltpu.sync_copy(rows0, acc.at[dst_v.at[r]], add=True)
                d1.wait()
                pltpu.sync_copy(rows1, acc.at[dst_v.at[r + 1]], add=True)
                return carry
            lax.fori_loop(0, CPS // 2, _pair, 0)
        plsc.subcore_barrier()

        pltpu.sync_copy(acc.at[pl.ds(s * RPT, RPT)],
                        out_hbm.at[c, pl.ds(s * RPT, RPT)])

    return k(xs, src, dst)


_R = 1000
_GRID = (N // _R,)


def _dinv_of(dp):
    return lax.rsqrt(1.0 + dp[0, :, 0] + dp[1, :, 0])[:, None]


def _row_spec():
    return pl.BlockSpec((_R, H), lambda i: (i, 0))


def _w_spec():
    return pl.BlockSpec((H, H), lambda i: (0, 0))


def _deg_spec():
    return pl.BlockSpec((NC, _R, 16), lambda i: (0, i, 0))


def _part_spec():
    return pl.BlockSpec((NC, _R, H), lambda i: (0, i, 0))


def _b_spec():
    return pl.BlockSpec((1, H), lambda i: (0, 0))


def _prep_body(x_ref, wt_ref, dp_ref, o_ref):
    dinv = _dinv_of(dp_ref[...])
    o_ref[...] = dinv * jnp.dot(x_ref[...], wt_ref[...],
                                preferred_element_type=F32)


def _tc_prep(x, wt, degp):
    return pl.pallas_call(
        _prep_body,
        grid=_GRID,
        in_specs=[_row_spec(), _w_spec(), _deg_spec()],
        out_specs=_row_spec(),
        out_shape=jax.ShapeDtypeStruct((N, H), F32),
    )(x, wt, degp)


def _mid_body(p_ref, xs_ref, dp_ref, b_ref, wt_ref, o_ref):
    dinv = _dinv_of(dp_ref[...])
    p = p_ref[...]
    h = jnp.maximum(dinv * (p[0] + p[1] + xs_ref[...]) + b_ref[...], 0.0)
    o_ref[...] = dinv * jnp.dot(h, wt_ref[...], preferred_element_type=F32)


def _tc_mid(parts, xs, degp, b, wt):
    return pl.pallas_call(
        _mid_body,
        grid=_GRID,
        in_specs=[_part_spec(), _row_spec(), _deg_spec(), _b_spec(),
                  _w_spec()],
        out_specs=_row_spec(),
        out_shape=jax.ShapeDtypeStruct((N, H), F32),
    )(parts, xs, degp, b, wt)


def _fin_body(pi_ref, xsi_ref, dpi_ref, bi_ref,
              ps_ref, xss_ref, dps_ref, bs_ref,
              fwi_ref, fws_ref, fb_ref, o_ref):
    dinv_i = _dinv_of(dpi_ref[...])
    pi = pi_ref[...]
    h_i = jnp.maximum(
        dinv_i * (pi[0] + pi[1] + xsi_ref[...]) + bi_ref[...], 0.0)
    dinv_s = _dinv_of(dps_ref[...])
    ps = ps_ref[...]
    h_s = jnp.maximum(
        dinv_s * (ps[0] + ps[1] + xss_ref[...]) + bs_ref[...], 0.0)
    o_ref[...] = (jnp.dot(h_i, fwi_ref[...], preferred_element_type=F32)
                  + jnp.dot(h_s, fws_ref[...], preferred_element_type=F32)
                  + fb_ref[...])


def _tc_fin(pi, xsi, dpi, bi, ps, xss, dps, bs, fwi, fws, fb):
    return pl.pallas_call(
        _fin_body,
        grid=_GRID,
        in_specs=[_part_spec(), _row_spec(), _deg_spec(), _b_spec(),
                  _part_spec(), _row_spec(), _deg_spec(), _b_spec(),
                  _w_spec(), _w_spec(), _b_spec()],
        out_specs=_row_spec(),
        out_shape=jax.ShapeDtypeStruct((N, H), F32),
    )(pi, xsi, dpi, bi, ps, xss, dps, bs, fwi, fws, fb)


def kernel(interaction_x, interaction_edge_index,
           similarity_x, similarity_edge_index,
           W_ic1, b_ic1, W_ic2, b_ic2,
           W_sc1, b_sc1, W_sc2, b_sc2, fc_W, fc_b):
    zpad = jnp.zeros((EP - E,), jnp.int32)
    npad = N + (jnp.arange(EP - E, dtype=jnp.int32) % (NP - N))
    src_i = jnp.concatenate(
        [interaction_edge_index[0], zpad]).reshape(NW, ST, CPS, K)
    dst_i = jnp.concatenate(
        [interaction_edge_index[1], npad]).reshape(NW, ST, CPS, K)
    src_s = jnp.concatenate(
        [similarity_edge_index[0], zpad]).reshape(NW, ST, CPS, K)
    dst_s = jnp.concatenate(
        [similarity_edge_index[1], npad]).reshape(NW, ST, CPS, K)
    dsts = jnp.stack([dst_i, dst_s])

    degp_i, degp_s = _sc_deg(dsts)

    b1_i = b_ic1.reshape(1, H)
    b2_i = b_ic2.reshape(1, H)
    b1_s = b_sc1.reshape(1, H)
    b2_s = b_sc2.reshape(1, H)
    fwt = fc_W.T
    fwt_i = fwt[:H]
    fwt_s = fwt[H:]
    fb = fc_b.reshape(1, H)

    xs1_i = _tc_prep(interaction_x, W_ic1.T, degp_i)
    xs1_s = _tc_prep(similarity_x, W_sc1.T, degp_s)
    p1_i = _sc_scatter(xs1_i, src_i, dst_i)
    p1_s = _sc_scatter(xs1_s, src_s, dst_s)
    xs2_i = _tc_mid(p1_i, xs1_i, degp_i, b1_i, W_ic2.T)
    xs2_s = _tc_mid(p1_s, xs1_s, degp_s, b1_s, W_sc2.T)
    p2_i = _sc_scatter(xs2_i, src_i, dst_i)
    p2_s = _sc_scatter(xs2_s, src_s, dst_s)
    return _tc_fin(p2_i, xs2_i, degp_i, b2_i,
                   p2_s, xs2_s, degp_s, b2_s,
                   fwt_i, fwt_s, fb)


# speedup vs baseline: 3.0678x; 3.0129x over previous
import functools

import jax
import jax.numpy as jnp
from jax import lax
from jax.experimental import pallas as pl
from jax.experimental.pallas import tpu as pltpu
from jax.experimental.pallas import tpu_sc as plsc

N = 10000
E = 320000
H = 128

NC = 2
NS = 16
NW = NC * NS
EP = 327680
EPT = EP // NW
K = 128
ST = 4
CPS = 20
NP = 10112
RPT = NP // NS

F32 = jnp.float32


def _mesh():
    return plsc.VectorSubcoreMesh(
        core_axis_name="c", subcore_axis_name="s",
        num_cores=NC, num_subcores=NS)


def _sc_deg(dsts):
    out_t = (jax.ShapeDtypeStruct((NC, NP, 16), F32),
             jax.ShapeDtypeStruct((NC, NP, 16), F32))

    @functools.partial(
        pl.kernel,
        out_type=out_t,
        mesh=_mesh(),
        scratch_types=[
            pltpu.VMEM((CPS, K), jnp.int32),
            pltpu.VMEM((K, 16), F32),
            pltpu.VMEM_SHARED((NP, 16), F32),
        ],
    )
    def k(dsts_hbm, oi_hbm, os_hbm, idx_v, ones_v, acc):
        c = lax.axis_index("c")
        s = lax.axis_index("s")
        wid = c * NS + s

        def _fill(val):
            def _f(i, carry):
                ones_v[i, :] = jnp.full((16,), val, F32)
                return carry
            lax.fori_loop(0, K, _f, 0)

        for g, out_hbm in enumerate((oi_hbm, os_hbm)):
            _fill(0.0)
            for j in range(RPT // K):
                pltpu.sync_copy(ones_v, acc.at[pl.ds(s * RPT + j * K, K)])
            pltpu.sync_copy(ones_v.at[pl.ds(0, RPT % K)],
                            acc.at[pl.ds(s * RPT + (RPT // K) * K, RPT % K)])
            plsc.subcore_barrier()
            _fill(1.0)

            for b in range(ST):
                pltpu.sync_copy(dsts_hbm.at[g, wid, b], idx_v)

                def _step(r, carry):
                    pltpu.sync_copy(ones_v, acc.at[idx_v.at[r]], add=True)
                    return carry
                lax.fori_loop(0, CPS, _step, 0)
            plsc.subcore_barrier()

            pltpu.sync_copy(acc.at[pl.ds(s * RPT, RPT)],
                            out_hbm.at[c, pl.ds(s * RPT, RPT)])

    return k(dsts)


def _sc_scatter(xs, src, dst):
    @functools.partial(
        pl.kernel,
        out_type=jax.ShapeDtypeStruct((NC, NP, H), F32),
        mesh=_mesh(),
        scratch_types=[
            pltpu.VMEM((CPS, K), jnp.int32),
            pltpu.VMEM((CPS, K), jnp.int32),
            pltpu.VMEM((K, H), F32),
            pltpu.VMEM((K, H), F32),
            pltpu.VMEM_SHARED((NP, H), F32),
            pltpu.SemaphoreType.DMA,
            pltpu.SemaphoreType.DMA,
        ],
    )
    def k(xs_hbm, src_hbm, dst_hbm, out_hbm,
          src_v, dst_v, rows0, rows1, acc, sem0, sem1):
        c = lax.axis_index("c")
        s = lax.axis_index("s")
        wid = c * NS + s
        base = s * RPT

        def _zrow(i, carry):
            for j in range(H // 16):
                rows0[i, pl.ds(j * 16, 16)] = jnp.zeros((16,), F32)
                rows1[i, pl.ds(j * 16, 16)] = jnp.zeros((16,), F32)
            return carry
        lax.fori_loop(0, K, _zrow, 0)
        for j in range(RPT // (2 * K)):
            pltpu.sync_copy(rows0, acc.at[pl.ds(base + j * 2 * K, K)])
            pltpu.sync_copy(rows1, acc.at[pl.ds(base + j * 2 * K + K, K)])
        pltpu.sync_copy(rows0.at[pl.ds(0, RPT % (2 * K))],
                        acc.at[pl.ds(base + (RPT // (2 * K)) * 2 * K,
                                     RPT % (2 * K))])
        plsc.subcore_barrier()

        for b in range(ST):
            pltpu.sync_copy(src_hbm.at[wid, b], src_v)
            pltpu.sync_copy(dst_hbm.at[wid, b], dst_v)

            def _pair(i, carry):
                r = 2 * i
                d0 = pltpu.async_copy(xs_hbm.at[src_v.at[r]], rows0, sem0)
                d1 = pltpu.async_copy(xs_hbm.at[src_v.at[r + 1]], rows1, sem1)
                d0.wait()
                pltpu.sync_copy(rows0, acc.at[dst_v.at[r]], add=True)
                d1.wait()
                pltpu.sync_copy(rows1, acc.at[dst_v.at[r + 1]], add=True)
                return carry
            lax.fori_loop(0, CPS // 2, _pair, 0)
        plsc.subcore_barrier()

        pltpu.sync_copy(acc.at[pl.ds(s * RPT, RPT)],
                        out_hbm.at[c, pl.ds(s * RPT, RPT)])

    return k(xs, src, dst)


_R = 1000
_GRID = (N // _R,)


def _dinv_of(dp):
    return lax.rsqrt(1.0 + dp[0, :, 0] + dp[1, :, 0])[:, None]


def _row_spec():
    return pl.BlockSpec((_R, H), lambda i: (i, 0))


def _w_spec():
    return pl.BlockSpec((H, H), lambda i: (0, 0))


def _deg_spec():
    return pl.BlockSpec((NC, _R, 16), lambda i: (0, i, 0))


def _part_spec():
    return pl.BlockSpec((NC, _R, H), lambda i: (0, i, 0))


def _b_spec():
    return pl.BlockSpec((1, H), lambda i: (0, 0))


def _prep_body(x_ref, wt_ref, dp_ref, o_ref):
    dinv = _dinv_of(dp_ref[...])
    o_ref[...] = dinv * jnp.dot(x_ref[...], wt_ref[...],
                                preferred_element_type=F32)


def _tc_prep(x, wt, degp):
    return pl.pallas_call(
        _prep_body,
        grid=_GRID,
        in_specs=[_row_spec(), _w_spec(), _deg_spec()],
        out_specs=_row_spec(),
        out_shape=jax.ShapeDtypeStruct((N, H), F32),
    )(x, wt, degp)


def _mid_body(p_ref, xs_ref, dp_ref, b_ref, wt_ref, o_ref):
    dinv = _dinv_of(dp_ref[...])
    p = p_ref[...]
    h = jnp.maximum(dinv * (p[0] + p[1] + xs_ref[...]) + b_ref[...], 0.0)
    o_ref[...] = dinv * jnp.dot(h, wt_ref[...], preferred_element_type=F32)


def _tc_mid(parts, xs, degp, b, wt):
    return pl.pallas_call(
        _mid_body,
        grid=_GRID,
        in_specs=[_part_spec(), _row_spec(), _deg_spec(), _b_spec(),
                  _w_spec()],
        out_specs=_row_spec(),
        out_shape=jax.ShapeDtypeStruct((N, H), F32),
    )(parts, xs, degp, b, wt)


def _fin_body(pi_ref, xsi_ref, dpi_ref, bi_ref,
              ps_ref, xss_ref, dps_ref, bs_ref,
              fwi_ref, fws_ref, fb_ref, o_ref):
    dinv_i = _dinv_of(dpi_ref[...])
    pi = pi_ref[...]
    h_i = jnp.maximum(
        dinv_i * (pi[0] + pi[1] + xsi_ref[...]) + bi_ref[...], 0.0)
    dinv_s = _dinv_of(dps_ref[...])
    ps = ps_ref[...]
    h_s = jnp.maximum(
        dinv_s * (ps[0] + ps[1] + xss_ref[...]) + bs_ref[...], 0.0)
    o_ref[...] = (jnp.dot(h_i, fwi_ref[...], preferred_element_type=F32)
                  + jnp.dot(h_s, fws_ref[...], preferred_element_type=F32)
                  + fb_ref[...])


def _tc_fin(pi, xsi, dpi, bi, ps, xss, dps, bs, fwi, fws, fb):
    return pl.pallas_call(
        _fin_body,
        grid=_GRID,
        in_specs=[_part_spec(), _row_spec(), _deg_spec(), _b_spec(),
                  _part_spec(), _row_spec(), _deg_spec(), _b_spec(),
                  _w_spec(), _w_spec(), _b_spec()],
        out_specs=_row_spec(),
        out_shape=jax.ShapeDtypeStruct((N, H), F32),
    )(pi, xsi, dpi, bi, ps, xss, dps, bs, fwi, fws, fb)


def kernel(interaction_x, interaction_edge_index,
           similarity_x, similarity_edge_index,
           W_ic1, b_ic1, W_ic2, b_ic2,
           W_sc1, b_sc1, W_sc2, b_sc2, fc_W, fc_b):
    zpad = jnp.arange(EP - E, dtype=jnp.int32) % N
    npad = N + (jnp.arange(EP - E, dtype=jnp.int32) % (NP - N))
    src_i = jnp.concatenate(
        [interaction_edge_index[0], zpad]).reshape(NW, ST, CPS, K)
    dst_i = jnp.concatenate(
        [interaction_edge_index[1], npad]).reshape(NW, ST, CPS, K)
    src_s = jnp.concatenate(
        [similarity_edge_index[0], zpad]).reshape(NW, ST, CPS, K)
    dst_s = jnp.concatenate(
        [similarity_edge_index[1], npad]).reshape(NW, ST, CPS, K)
    dsts = jnp.stack([dst_i, dst_s])

    degp_i, degp_s = _sc_deg(dsts)

    b1_i = b_ic1.reshape(1, H)
    b2_i = b_ic2.reshape(1, H)
    b1_s = b_sc1.reshape(1, H)
    b2_s = b_sc2.reshape(1, H)
    fwt = fc_W.T
    fwt_i = fwt[:H]
    fwt_s = fwt[H:]
    fb = fc_b.reshape(1, H)

    xs1_i = _tc_prep(interaction_x, W_ic1.T, degp_i)
    xs1_s = _tc_prep(similarity_x, W_sc1.T, degp_s)
    p1_i = _sc_scatter(xs1_i, src_i, dst_i)
    p1_s = _sc_scatter(xs1_s, src_s, dst_s)
    xs2_i = _tc_mid(p1_i, xs1_i, degp_i, b1_i, W_ic2.T)
    xs2_s = _tc_mid(p1_s, xs1_s, degp_s, b1_s, W_sc2.T)
    p2_i = _sc_scatter(xs2_i, src_i, dst_i)
    p2_s = _sc_scatter(xs2_s, src_s, dst_s)
    return _tc_fin(p2_i, xs2_i, degp_i, b2_i,
                   p2_s, xs2_s, degp_s, b2_s,
                   fwt_i, fwt_s, fb)


# speedup vs baseline: 3.6609x; 1.1933x over previous
import functools

import jax
import jax.numpy as jnp
from jax import lax
from jax.experimental import pallas as pl
from jax.experimental.pallas import tpu as pltpu
from jax.experimental.pallas import tpu_sc as plsc

N = 10000
E = 320000
H = 128

NC = 2
NS = 16
NW = NC * NS
EP = 327680
EPT = EP // NW
K = 80
ST = 4
CPS = 32
NBUF = 4
NP = 10112
RPT = NP // NS

F32 = jnp.float32


def _mesh():
    return plsc.VectorSubcoreMesh(
        core_axis_name="c", subcore_axis_name="s",
        num_cores=NC, num_subcores=NS)


def _sc_deg(dsts):
    out_t = (jax.ShapeDtypeStruct((NC, NP, 16), F32),
             jax.ShapeDtypeStruct((NC, NP, 16), F32))

    @functools.partial(
        pl.kernel,
        out_type=out_t,
        mesh=_mesh(),
        scratch_types=[
            pltpu.VMEM((CPS, K), jnp.int32),
            pltpu.VMEM((K, 16), F32),
            pltpu.VMEM_SHARED((NP, 16), F32),
        ],
    )
    def k(dsts_hbm, oi_hbm, os_hbm, idx_v, ones_v, acc):
        c = lax.axis_index("c")
        s = lax.axis_index("s")
        wid = c * NS + s

        def _fill(val):
            def _f(i, carry):
                ones_v[i, :] = jnp.full((16,), val, F32)
                return carry
            lax.fori_loop(0, K, _f, 0)

        for g, out_hbm in enumerate((oi_hbm, os_hbm)):
            _fill(0.0)
            for j in range(RPT // K):
                pltpu.sync_copy(ones_v, acc.at[pl.ds(s * RPT + j * K, K)])
            pltpu.sync_copy(ones_v.at[pl.ds(0, RPT % K)],
                            acc.at[pl.ds(s * RPT + (RPT // K) * K, RPT % K)])
            plsc.subcore_barrier()
            _fill(1.0)

            for b in range(ST):
                pltpu.sync_copy(dsts_hbm.at[g, wid, b], idx_v)

                def _step(r, carry):
                    pltpu.sync_copy(ones_v, acc.at[idx_v.at[r]], add=True)
                    return carry
                lax.fori_loop(0, CPS, _step, 0)
            plsc.subcore_barrier()

            pltpu.sync_copy(acc.at[pl.ds(s * RPT, RPT)],
                            out_hbm.at[c, pl.ds(s * RPT, RPT)])

    return k(dsts)


def _sc_scatter(xs, src, dst):
    @functools.partial(
        pl.kernel,
        out_type=jax.ShapeDtypeStruct((NC, NP, H), F32),
        mesh=_mesh(),
        scratch_types=[
            pltpu.VMEM((CPS, K), jnp.int32),
            pltpu.VMEM((CPS, K), jnp.int32),
            pltpu.VMEM((NBUF, K, H), F32),
            pltpu.VMEM_SHARED((NP, H), F32),
            pltpu.SemaphoreType.DMA,
            pltpu.SemaphoreType.DMA,
            pltpu.SemaphoreType.DMA,
            pltpu.SemaphoreType.DMA,
            pltpu.SemaphoreType.DMA,
            pltpu.SemaphoreType.DMA,
            pltpu.SemaphoreType.DMA,
            pltpu.SemaphoreType.DMA,
        ],
    )
    def k(xs_hbm, src_hbm, dst_hbm, out_hbm,
          src_v, dst_v, rows, acc,
          g0, g1, g2, g3, s0, s1, s2, s3):
        gs = (g0, g1, g2, g3)
        ss = (s0, s1, s2, s3)
        c = lax.axis_index("c")
        s = lax.axis_index("s")
        wid = c * NS + s
        base = s * RPT

        def _gwait(q):
            pltpu.make_async_copy(
                xs_hbm.at[pl.ds(0, K)], rows.at[q], gs[q]).wait()

        def _swait(q):
            pltpu.make_async_copy(
                rows.at[q], acc.at[pl.ds(0, K)], ss[q]).wait()

        def _zrow(i, carry):
            for q in range(NBUF):
                for j in range(H // 16):
                    rows[q, i, pl.ds(j * 16, 16)] = jnp.zeros((16,), F32)
            return carry
        lax.fori_loop(0, K, _zrow, 0)
        for j in range(RPT // K):
            pltpu.sync_copy(rows.at[j % NBUF],
                            acc.at[pl.ds(base + j * K, K)])
        pltpu.sync_copy(rows.at[NBUF - 1, pl.ds(0, RPT % K)],
                        acc.at[pl.ds(base + (RPT // K) * K, RPT % K)])
        plsc.subcore_barrier()

        for b in range(ST):
            pltpu.sync_copy(src_hbm.at[wid, b], src_v)
            pltpu.sync_copy(dst_hbm.at[wid, b], dst_v)
            for q in range(NBUF):
                pltpu.async_copy(xs_hbm.at[src_v.at[q]], rows.at[q], gs[q])

            def _round(j, carry):
                for q in range(NBUF):
                    r = j * NBUF + q
                    _gwait(q)
                    pltpu.async_copy(rows.at[q], acc.at[dst_v.at[r]],
                                     ss[q], add=True)
                for q in range(NBUF):
                    r2 = (j + 1) * NBUF + q
                    _swait(q)
                    pltpu.async_copy(xs_hbm.at[src_v.at[r2]], rows.at[q],
                                     gs[q])
                return carry
            lax.fori_loop(0, CPS // NBUF - 1, _round, 0)

            for q in range(NBUF):
                r = CPS - NBUF + q
                _gwait(q)
                pltpu.async_copy(rows.at[q], acc.at[dst_v.at[r]],
                                 ss[q], add=True)
            for q in range(NBUF):
                _swait(q)
        plsc.subcore_barrier()

        pltpu.sync_copy(acc.at[pl.ds(s * RPT, RPT)],
                        out_hbm.at[c, pl.ds(s * RPT, RPT)])

    return k(xs, src, dst)


_R = 1000
_GRID = (N // _R,)


def _dinv_of(dp):
    return lax.rsqrt(1.0 + dp[0, :, 0] + dp[1, :, 0])[:, None]


def _row_spec():
    return pl.BlockSpec((_R, H), lambda i: (i, 0))


def _w_spec():
    return pl.BlockSpec((H, H), lambda i: (0, 0))


def _deg_spec():
    return pl.BlockSpec((NC, _R, 16), lambda i: (0, i, 0))


def _part_spec():
    return pl.BlockSpec((NC, _R, H), lambda i: (0, i, 0))


def _b_spec():
    return pl.BlockSpec((1, H), lambda i: (0, 0))


def _prep_body(x_ref, wt_ref, dp_ref, o_ref):
    dinv = _dinv_of(dp_ref[...])
    o_ref[...] = dinv * jnp.dot(x_ref[...], wt_ref[...],
                                preferred_element_type=F32)


def _tc_prep(x, wt, degp):
    return pl.pallas_call(
        _prep_body,
        grid=_GRID,
        in_specs=[_row_spec(), _w_spec(), _deg_spec()],
        out_specs=_row_spec(),
        out_shape=jax.ShapeDtypeStruct((N, H), F32),
    )(x, wt, degp)


def _mid_body(p_ref, xs_ref, dp_ref, b_ref, wt_ref, o_ref):
    dinv = _dinv_of(dp_ref[...])
    p = p_ref[...]
    h = jnp.maximum(dinv * (p[0] + p[1] + xs_ref[...]) + b_ref[...], 0.0)
    o_ref[...] = dinv * jnp.dot(h, wt_ref[...], preferred_element_type=F32)


def _tc_mid(parts, xs, degp, b, wt):
    return pl.pallas_call(
        _mid_body,
        grid=_GRID,
        in_specs=[_part_spec(), _row_spec(), _deg_spec(), _b_spec(),
                  _w_spec()],
        out_specs=_row_spec(),
        out_shape=jax.ShapeDtypeStruct((N, H), F32),
    )(parts, xs, degp, b, wt)


def _fin_body(pi_ref, xsi_ref, dpi_ref, bi_ref,
              ps_ref, xss_ref, dps_ref, bs_ref,
              fwi_ref, fws_ref, fb_ref, o_ref):
    dinv_i = _dinv_of(dpi_ref[...])
    pi = pi_ref[...]
    h_i = jnp.maximum(
        dinv_i * (pi[0] + pi[1] + xsi_ref[...]) + bi_ref[...], 0.0)
    dinv_s = _dinv_of(dps_ref[...])
    ps = ps_ref[...]
    h_s = jnp.maximum(
        dinv_s * (ps[0] + ps[1] + xss_ref[...]) + bs_ref[...], 0.0)
    o_ref[...] = (jnp.dot(h_i, fwi_ref[...], preferred_element_type=F32)
                  + jnp.dot(h_s, fws_ref[...], preferred_element_type=F32)
                  + fb_ref[...])


def _tc_fin(pi, xsi, dpi, bi, ps, xss, dps, bs, fwi, fws, fb):
    return pl.pallas_call(
        _fin_body,
        grid=_GRID,
        in_specs=[_part_spec(), _row_spec(), _deg_spec(), _b_spec(),
                  _part_spec(), _row_spec(), _deg_spec(), _b_spec(),
                  _w_spec(), _w_spec(), _b_spec()],
        out_specs=_row_spec(),
        out_shape=jax.ShapeDtypeStruct((N, H), F32),
    )(pi, xsi, dpi, bi, ps, xss, dps, bs, fwi, fws, fb)


def kernel(interaction_x, interaction_edge_index,
           similarity_x, similarity_edge_index,
           W_ic1, b_ic1, W_ic2, b_ic2,
           W_sc1, b_sc1, W_sc2, b_sc2, fc_W, fc_b):
    zpad = jnp.arange(EP - E, dtype=jnp.int32) % N
    npad = N + (jnp.arange(EP - E, dtype=jnp.int32) % (NP - N))
    src_i = jnp.concatenate(
        [interaction_edge_index[0], zpad]).reshape(NW, ST, CPS, K)
    dst_i = jnp.concatenate(
        [interaction_edge_index[1], npad]).reshape(NW, ST, CPS, K)
    src_s = jnp.concatenate(
        [similarity_edge_index[0], zpad]).reshape(NW, ST, CPS, K)
    dst_s = jnp.concatenate(
        [similarity_edge_index[1], npad]).reshape(NW, ST, CPS, K)
    dsts = jnp.stack([dst_i, dst_s])

    degp_i, degp_s = _sc_deg(dsts)

    b1_i = b_ic1.reshape(1, H)
    b2_i = b_ic2.reshape(1, H)
    b1_s = b_sc1.reshape(1, H)
    b2_s = b_sc2.reshape(1, H)
    fwt = fc_W.T
    fwt_i = fwt[:H]
    fwt_s = fwt[H:]
    fb = fc_b.reshape(1, H)

    xs1_i = _tc_prep(interaction_x, W_ic1.T, degp_i)
    xs1_s = _tc_prep(similarity_x, W_sc1.T, degp_s)
    p1_i = _sc_scatter(xs1_i, src_i, dst_i)
    p1_s = _sc_scatter(xs1_s, src_s, dst_s)
    xs2_i = _tc_mid(p1_i, xs1_i, degp_i, b1_i, W_ic2.T)
    xs2_s = _tc_mid(p1_s, xs1_s, degp_s, b1_s, W_sc2.T)
    p2_i = _sc_scatter(xs2_i, src_i, dst_i)
    p2_s = _sc_scatter(xs2_s, src_s, dst_s)
    return _tc_fin(p2_i, xs2_i, degp_i, b2_i,
                   p2_s, xs2_s, degp_s, b2_s,
                   fwt_i, fwt_s, fb)


# speedup vs baseline: 3.6691x; 1.0022x over previous
import functools

import jax
import jax.numpy as jnp
from jax import lax
from jax.experimental import pallas as pl
from jax.experimental.pallas import tpu as pltpu
from jax.experimental.pallas import tpu_sc as plsc

N = 10000
E = 320000
H = 128

NC = 2
NS = 16
NW = NC * NS
EP = 327680
EPT = EP // NW
K = 80
ST = 4
CPS = 32
NBUF = 4
NP = 10112
RPT = NP // NS

F32 = jnp.float32


def _mesh():
    return plsc.VectorSubcoreMesh(
        core_axis_name="c", subcore_axis_name="s",
        num_cores=NC, num_subcores=NS)


def _sc_deg(dsts):
    out_t = (jax.ShapeDtypeStruct((NC, NP, 16), F32),
             jax.ShapeDtypeStruct((NC, NP, 16), F32))

    @functools.partial(
        pl.kernel,
        out_type=out_t,
        mesh=_mesh(),
        scratch_types=[
            pltpu.VMEM((CPS, K), jnp.int32),
            pltpu.VMEM((K, 16), F32),
            pltpu.VMEM_SHARED((NP, 16), F32),
        ],
    )
    def k(dsts_hbm, oi_hbm, os_hbm, idx_v, ones_v, acc):
        c = lax.axis_index("c")
        s = lax.axis_index("s")
        wid = c * NS + s

        def _fill(val):
            def _f(i, carry):
                ones_v[i, :] = jnp.full((16,), val, F32)
                return carry
            lax.fori_loop(0, K, _f, 0)

        for g, out_hbm in enumerate((oi_hbm, os_hbm)):
            _fill(0.0)
            for j in range(RPT // K):
                pltpu.sync_copy(ones_v, acc.at[pl.ds(s * RPT + j * K, K)])
            pltpu.sync_copy(ones_v.at[pl.ds(0, RPT % K)],
                            acc.at[pl.ds(s * RPT + (RPT // K) * K, RPT % K)])
            plsc.subcore_barrier()
            _fill(1.0)

            for b in range(ST):
                pltpu.sync_copy(dsts_hbm.at[g, wid, b], idx_v)

                def _step(r, carry):
                    pltpu.sync_copy(ones_v, acc.at[idx_v.at[r]], add=True)
                    return carry
                lax.fori_loop(0, CPS, _step, 0)
            plsc.subcore_barrier()

            pltpu.sync_copy(acc.at[pl.ds(s * RPT, RPT)],
                            out_hbm.at[c, pl.ds(s * RPT, RPT)])

    return k(dsts)


def _sc_scatter(xs, src, dst):
    @functools.partial(
        pl.kernel,
        out_type=jax.ShapeDtypeStruct((NC, NP, H), F32),
        mesh=_mesh(),
        scratch_types=[
            pltpu.VMEM((CPS, K), jnp.int32),
            pltpu.VMEM((CPS, K), jnp.int32),
            pltpu.VMEM((NBUF, K, H), F32),
            pltpu.VMEM_SHARED((NP, H), F32),
            pltpu.SemaphoreType.DMA,
            pltpu.SemaphoreType.DMA,
            pltpu.SemaphoreType.DMA,
            pltpu.SemaphoreType.DMA,
            pltpu.SemaphoreType.DMA,
            pltpu.SemaphoreType.DMA,
            pltpu.SemaphoreType.DMA,
            pltpu.SemaphoreType.DMA,
        ],
    )
    def k(xs_hbm, src_hbm, dst_hbm, out_hbm,
          src_v, dst_v, rows, acc,
          g0, g1, g2, g3, s0, s1, s2, s3):
        gs = (g0, g1, g2, g3)
        ss = (s0, s1, s2, s3)
        c = lax.axis_index("c")
        s = lax.axis_index("s")
        wid = c * NS + s
        base = s * RPT

        def _gwait(q, r):
            pltpu.make_async_copy(
                xs_hbm.at[src_v.at[r]], rows.at[q], gs[q]).wait()

        def _swait(q, r):
            pltpu.make_async_copy(
                rows.at[q], acc.at[dst_v.at[r]], ss[q]).wait()

        def _zrow(i, carry):
            for q in range(NBUF):
                for j in range(H // 16):
                    rows[q, i, pl.ds(j * 16, 16)] = jnp.zeros((16,), F32)
            return carry
        lax.fori_loop(0, K, _zrow, 0)
        for j in range(RPT // K):
            pltpu.sync_copy(rows.at[j % NBUF],
                            acc.at[pl.ds(base + j * K, K)])
        pltpu.sync_copy(rows.at[NBUF - 1, pl.ds(0, RPT % K)],
                        acc.at[pl.ds(base + (RPT // K) * K, RPT % K)])
        plsc.subcore_barrier()

        for b in range(ST):
            pltpu.sync_copy(src_hbm.at[wid, b], src_v)
            pltpu.sync_copy(dst_hbm.at[wid, b], dst_v)
            for q in range(NBUF):
                pltpu.async_copy(xs_hbm.at[src_v.at[q]], rows.at[q], gs[q])

            def _round(j, carry):
                for q in range(NBUF):
                    r = j * NBUF + q
                    _gwait(q, r)
                    pltpu.async_copy(rows.at[q], acc.at[dst_v.at[r]],
                                     ss[q], add=True)
                for q in range(NBUF):
                    r = j * NBUF + q
                    _swait(q, r)
                    pltpu.async_copy(xs_hbm.at[src_v.at[r + NBUF]],
                                     rows.at[q], gs[q])
                return carry
            lax.fori_loop(0, CPS // NBUF - 1, _round, 0)

            for q in range(NBUF):
                r = CPS - NBUF + q
                _gwait(q, r)
                pltpu.async_copy(rows.at[q], acc.at[dst_v.at[r]],
                                 ss[q], add=True)
            for q in range(NBUF):
                _swait(q, CPS - NBUF + q)
        plsc.subcore_barrier()

        pltpu.sync_copy(acc.at[pl.ds(s * RPT, RPT)],
                        out_hbm.at[c, pl.ds(s * RPT, RPT)])

    return k(xs, src, dst)


_R = 1000
_GRID = (N // _R,)


def _dinv_of(dp):
    return lax.rsqrt(1.0 + dp[0, :, 0] + dp[1, :, 0])[:, None]


def _row_spec():
    return pl.BlockSpec((_R, H), lambda i: (i, 0))


def _w_spec():
    return pl.BlockSpec((H, H), lambda i: (0, 0))


def _deg_spec():
    return pl.BlockSpec((NC, _R, 16), lambda i: (0, i, 0))


def _part_spec():
    return pl.BlockSpec((NC, _R, H), lambda i: (0, i, 0))


def _b_spec():
    return pl.BlockSpec((1, H), lambda i: (0, 0))


def _prep_body(x_ref, wt_ref, dp_ref, o_ref):
    dinv = _dinv_of(dp_ref[...])
    o_ref[...] = dinv * jnp.dot(x_ref[...], wt_ref[...],
                                preferred_element_type=F32)


def _tc_prep(x, wt, degp):
    return pl.pallas_call(
        _prep_body,
        grid=_GRID,
        in_specs=[_row_spec(), _w_spec(), _deg_spec()],
        out_specs=_row_spec(),
        out_shape=jax.ShapeDtypeStruct((N, H), F32),
    )(x, wt, degp)


def _mid_body(p_ref, xs_ref, dp_ref, b_ref, wt_ref, o_ref):
    dinv = _dinv_of(dp_ref[...])
    p = p_ref[...]
    h = jnp.maximum(dinv * (p[0] + p[1] + xs_ref[...]) + b_ref[...], 0.0)
    o_ref[...] = dinv * jnp.dot(h, wt_ref[...], preferred_element_type=F32)


def _tc_mid(parts, xs, degp, b, wt):
    return pl.pallas_call(
        _mid_body,
        grid=_GRID,
        in_specs=[_part_spec(), _row_spec(), _deg_spec(), _b_spec(),
                  _w_spec()],
        out_specs=_row_spec(),
        out_shape=jax.ShapeDtypeStruct((N, H), F32),
    )(parts, xs, degp, b, wt)


def _fin_body(pi_ref, xsi_ref, dpi_ref, bi_ref,
              ps_ref, xss_ref, dps_ref, bs_ref,
              fwi_ref, fws_ref, fb_ref, o_ref):
    dinv_i = _dinv_of(dpi_ref[...])
    pi = pi_ref[...]
    h_i = jnp.maximum(
        dinv_i * (pi[0] + pi[1] + xsi_ref[...]) + bi_ref[...], 0.0)
    dinv_s = _dinv_of(dps_ref[...])
    ps = ps_ref[...]
    h_s = jnp.maximum(
        dinv_s * (ps[0] + ps[1] + xss_ref[...]) + bs_ref[...], 0.0)
    o_ref[...] = (jnp.dot(h_i, fwi_ref[...], preferred_element_type=F32)
                  + jnp.dot(h_s, fws_ref[...], preferred_element_type=F32)
                  + fb_ref[...])


def _tc_fin(pi, xsi, dpi, bi, ps, xss, dps, bs, fwi, fws, fb):
    return pl.pallas_call(
        _fin_body,
        grid=_GRID,
        in_specs=[_part_spec(), _row_spec(), _deg_spec(), _b_spec(),
                  _part_spec(), _row_spec(), _deg_spec(), _b_spec(),
                  _w_spec(), _w_spec(), _b_spec()],
        out_specs=_row_spec(),
        out_shape=jax.ShapeDtypeStruct((N, H), F32),
    )(pi, xsi, dpi, bi, ps, xss, dps, bs, fwi, fws, fb)


def kernel(interaction_x, interaction_edge_index,
           similarity_x, similarity_edge_index,
           W_ic1, b_ic1, W_ic2, b_ic2,
           W_sc1, b_sc1, W_sc2, b_sc2, fc_W, fc_b):
    zpad = jnp.arange(EP - E, dtype=jnp.int32) % N
    npad = N + (jnp.arange(EP - E, dtype=jnp.int32) % (NP - N))
    src_i = jnp.concatenate(
        [interaction_edge_index[0], zpad]).reshape(NW, ST, CPS, K)
    dst_i = jnp.concatenate(
        [interaction_edge_index[1], npad]).reshape(NW, ST, CPS, K)
    src_s = jnp.concatenate(
        [similarity_edge_index[0], zpad]).reshape(NW, ST, CPS, K)
    dst_s = jnp.concatenate(
        [similarity_edge_index[1], npad]).reshape(NW, ST, CPS, K)
    dsts = jnp.stack([dst_i, dst_s])

    degp_i, degp_s = _sc_deg(dsts)

    b1_i = b_ic1.reshape(1, H)
    b2_i = b_ic2.reshape(1, H)
    b1_s = b_sc1.reshape(1, H)
    b2_s = b_sc2.reshape(1, H)
    fwt = fc_W.T
    fwt_i = fwt[:H]
    fwt_s = fwt[H:]
    fb = fc_b.reshape(1, H)

    xs1_i = _tc_prep(interaction_x, W_ic1.T, degp_i)
    xs1_s = _tc_prep(similarity_x, W_sc1.T, degp_s)
    p1_i = _sc_scatter(xs1_i, src_i, dst_i)
    p1_s = _sc_scatter(xs1_s, src_s, dst_s)
    xs2_i = _tc_mid(p1_i, xs1_i, degp_i, b1_i, W_ic2.T)
    xs2_s = _tc_mid(p1_s, xs1_s, degp_s, b1_s, W_sc2.T)
    p2_i = _sc_scatter(xs2_i, src_i, dst_i)
    p2_s = _sc_scatter(xs2_s, src_s, dst_s)
    return _tc_fin(p2_i, xs2_i, degp_i, b2_i,
                   p2_s, xs2_s, degp_s, b2_s,
                   fwt_i, fwt_s, fb)


# speedup vs baseline: 3.8844x; 1.0587x over previous
import functools

import jax
import jax.numpy as jnp
from jax import lax
from jax.experimental import pallas as pl
from jax.experimental.pallas import tpu as pltpu
from jax.experimental.pallas import tpu_sc as plsc

N = 10000
E = 320000
H = 128

NC = 2
NS = 16
EP = 327680
EPT = EP // NS
K = 80
ST = 8
CPS = 32
NBUF = 4
DW = 8
NP = 10112
RPT = NP // NS

F32 = jnp.float32


def _mesh():
    return plsc.VectorSubcoreMesh(
        core_axis_name="c", subcore_axis_name="s",
        num_cores=NC, num_subcores=NS)


def _sc_deg(dsts):
    @functools.partial(
        pl.kernel,
        out_type=jax.ShapeDtypeStruct((NC, NP, 16), F32),
        mesh=_mesh(),
        scratch_types=[
            pltpu.VMEM((CPS, K), jnp.int32),
            pltpu.VMEM((K, 16), F32),
            pltpu.VMEM_SHARED((NP, 16), F32),
            pltpu.SemaphoreType.DMA,
        ],
    )
    def k(dsts_hbm, out_hbm, idx_v, ones_v, acc, sem):
        c = lax.axis_index("c")
        s = lax.axis_index("s")
        base = s * RPT

        def _fill(val):
            def _f(i, carry):
                ones_v[i, :] = jnp.full((16,), val, F32)
                return carry
            lax.fori_loop(0, K, _f, 0)

        _fill(0.0)
        for j in range(RPT // K):
            pltpu.sync_copy(ones_v, acc.at[pl.ds(base + j * K, K)])
        pltpu.sync_copy(ones_v.at[pl.ds(0, RPT % K)],
                        acc.at[pl.ds(base + (RPT // K) * K, RPT % K)])
        plsc.subcore_barrier()
        _fill(1.0)

        def _fire(r):
            pltpu.async_copy(ones_v, acc.at[idx_v.at[r]], sem, add=True)

        def _drain(r):
            pltpu.make_async_copy(ones_v, acc.at[idx_v.at[r]], sem).wait()

        for b in range(ST):
            pltpu.sync_copy(dsts_hbm.at[c, s, b], idx_v)
            for q in range(DW):
                _fire(q)

            def _step(j, carry):
                _drain(j)
                _fire(j + DW)
                return carry
            lax.fori_loop(0, CPS - DW, _step, 0)
            for q in range(CPS - DW, CPS):
                _drain(q)
        plsc.subcore_barrier()

        pltpu.sync_copy(acc.at[pl.ds(base, RPT)],
                        out_hbm.at[c, pl.ds(base, RPT)])

    return k(dsts)


def _sc_scatter(xs, src, dst):
    @functools.partial(
        pl.kernel,
        out_type=jax.ShapeDtypeStruct((NC, NP, H), F32),
        mesh=_mesh(),
        scratch_types=[
            pltpu.VMEM((CPS, K), jnp.int32),
            pltpu.VMEM((CPS, K), jnp.int32),
            pltpu.VMEM((NBUF, K, H), F32),
            pltpu.VMEM_SHARED((NP, H), F32),
            pltpu.SemaphoreType.DMA,
            pltpu.SemaphoreType.DMA,
            pltpu.SemaphoreType.DMA,
            pltpu.SemaphoreType.DMA,
            pltpu.SemaphoreType.DMA,
            pltpu.SemaphoreType.DMA,
            pltpu.SemaphoreType.DMA,
            pltpu.SemaphoreType.DMA,
        ],
    )
    def k(xs_hbm, src_hbm, dst_hbm, out_hbm,
          src_v, dst_v, rows, acc,
          g0, g1, g2, g3, s0, s1, s2, s3):
        gs = (g0, g1, g2, g3)
        ss = (s0, s1, s2, s3)
        c = lax.axis_index("c")
        s = lax.axis_index("s")
        base = s * RPT

        def _gwait(q, r):
            pltpu.make_async_copy(
                xs_hbm.at[src_v.at[r]], rows.at[q], gs[q]).wait()

        def _swait(q, r):
            pltpu.make_async_copy(
                rows.at[q], acc.at[dst_v.at[r]], ss[q]).wait()

        def _zrow(i, carry):
            for q in range(NBUF):
                for j in range(H // 16):
                    rows[q, i, pl.ds(j * 16, 16)] = jnp.zeros((16,), F32)
            return carry
        lax.fori_loop(0, K, _zrow, 0)
        for j in range(RPT // K):
            pltpu.sync_copy(rows.at[j % NBUF],
                            acc.at[pl.ds(base + j * K, K)])
        pltpu.sync_copy(rows.at[NBUF - 1, pl.ds(0, RPT % K)],
                        acc.at[pl.ds(base + (RPT // K) * K, RPT % K)])
        plsc.subcore_barrier()

        for b in range(ST):
            pltpu.sync_copy(src_hbm.at[c, s, b], src_v)
            pltpu.sync_copy(dst_hbm.at[c, s, b], dst_v)
            for q in range(NBUF):
                pltpu.async_copy(xs_hbm.at[src_v.at[q]], rows.at[q], gs[q])

            def _round(j, carry):
                for q in range(NBUF):
                    r = j * NBUF + q
                    _gwait(q, r)
                    pltpu.async_copy(rows.at[q], acc.at[dst_v.at[r]],
                                     ss[q], add=True)
                for q in range(NBUF):
                    r = j * NBUF + q
                    _swait(q, r)
                    pltpu.async_copy(xs_hbm.at[src_v.at[r + NBUF]],
                                     rows.at[q], gs[q])
                return carry
            lax.fori_loop(0, CPS // NBUF - 1, _round, 0)

            for q in range(NBUF):
                r = CPS - NBUF + q
                _gwait(q, r)
                pltpu.async_copy(rows.at[q], acc.at[dst_v.at[r]],
                                 ss[q], add=True)
            for q in range(NBUF):
                _swait(q, CPS - NBUF + q)
        plsc.subcore_barrier()

        pltpu.sync_copy(acc.at[pl.ds(base, RPT)],
                        out_hbm.at[c, pl.ds(base, RPT)])

    return k(xs, src, dst)


_R = 1000


def _g_row_spec():
    return pl.BlockSpec((1, _R, H), lambda g, i: (g, i, 0))


def _g_w_spec():
    return pl.BlockSpec((1, H, H), lambda g, i: (g, 0, 0))


def _g_deg_spec():
    return pl.BlockSpec((1, _R, 16), lambda g, i: (g, i, 0))


def _g_b_spec():
    return pl.BlockSpec((1, 1, H), lambda g, i: (g, 0, 0))


def _prep_body(x_ref, wt_ref, dg_ref, o_ref):
    dinv = lax.rsqrt(1.0 + dg_ref[0, :, 0])[:, None]
    o_ref[0] = dinv * jnp.dot(x_ref[0], wt_ref[0],
                              preferred_element_type=F32)


def _tc_prep(x, wt, degp):
    return pl.pallas_call(
        _prep_body,
        grid=(2, N // _R),
        in_specs=[_g_row_spec(), _g_w_spec(), _g_deg_spec()],
        out_specs=_g_row_spec(),
        out_shape=jax.ShapeDtypeStruct((2, N, H), F32),
    )(x, wt, degp)


def _mid_body(p_ref, xs_ref, dg_ref, b_ref, wt_ref, o_ref):
    dinv = lax.rsqrt(1.0 + dg_ref[0, :, 0])[:, None]
    h = jnp.maximum(dinv * (p_ref[0] + xs_ref[0]) + b_ref[0], 0.0)
    o_ref[0] = dinv * jnp.dot(h, wt_ref[0], preferred_element_type=F32)


def _tc_mid(parts, xs, degp, b, wt):
    return pl.pallas_call(
        _mid_body,
        grid=(2, N // _R),
        in_specs=[_g_row_spec(), _g_row_spec(), _g_deg_spec(),
                  _g_b_spec(), _g_w_spec()],
        out_specs=_g_row_spec(),
        out_shape=jax.ShapeDtypeStruct((2, N, H), F32),
    )(parts, xs, degp, b, wt)


def _fin_body(p_ref, xs_ref, dg_ref, b_ref, fwi_ref, fws_ref, fb_ref,
              o_ref):
    dinv_i = lax.rsqrt(1.0 + dg_ref[0, :, 0])[:, None]
    h_i = jnp.maximum(dinv_i * (p_ref[0] + xs_ref[0]) + b_ref[0], 0.0)
    dinv_s = lax.rsqrt(1.0 + dg_ref[1, :, 0])[:, None]
    h_s = jnp.maximum(dinv_s * (p_ref[1] + xs_ref[1]) + b_ref[1], 0.0)
    o_ref[...] = (jnp.dot(h_i, fwi_ref[...], preferred_element_type=F32)
                  + jnp.dot(h_s, fws_ref[...], preferred_element_type=F32)
                  + fb_ref[...])


def _tc_fin(parts, xs, degp, b, fwi, fws, fb):
    return pl.pallas_call(
        _fin_body,
        grid=(N // _R,),
        in_specs=[
            pl.BlockSpec((2, _R, H), lambda i: (0, i, 0)),
            pl.BlockSpec((2, _R, H), lambda i: (0, i, 0)),
            pl.BlockSpec((2, _R, 16), lambda i: (0, i, 0)),
            pl.BlockSpec((2, 1, H), lambda i: (0, 0, 0)),
            pl.BlockSpec((H, H), lambda i: (0, 0)),
            pl.BlockSpec((H, H), lambda i: (0, 0)),
            pl.BlockSpec((1, H), lambda i: (0, 0)),
        ],
        out_specs=pl.BlockSpec((_R, H), lambda i: (i, 0)),
        out_shape=jax.ShapeDtypeStruct((N, H), F32),
    )(parts, xs, degp, b, fwi, fws, fb)


def kernel(interaction_x, interaction_edge_index,
           similarity_x, similarity_edge_index,
           W_ic1, b_ic1, W_ic2, b_ic2,
           W_sc1, b_sc1, W_sc2, b_sc2, fc_W, fc_b):
    zpad = jnp.arange(EP - E, dtype=jnp.int32) % N
    npad = N + (jnp.arange(EP - E, dtype=jnp.int32) % (NP - N))
    shp = (NS, ST, CPS, K)
    src_i = jnp.concatenate([interaction_edge_index[0], zpad]).reshape(shp)
    dst_i = jnp.concatenate([interaction_edge_index[1], npad]).reshape(shp)
    src_s = (jnp.concatenate([similarity_edge_index[0], zpad]) + N
             ).reshape(shp)
    dst_s = jnp.concatenate([similarity_edge_index[1], npad]).reshape(shp)
    srcs = jnp.stack([src_i, src_s])
    dsts = jnp.stack([dst_i, dst_s])

    x_cat = jnp.stack([interaction_x, similarity_x])
    wt1 = jnp.stack([W_ic1.T, W_sc1.T])
    wt2 = jnp.stack([W_ic2.T, W_sc2.T])
    b1 = jnp.stack([b_ic1.reshape(1, H), b_sc1.reshape(1, H)])
    b2 = jnp.stack([b_ic2.reshape(1, H), b_sc2.reshape(1, H)])
    fwt = fc_W.T
    fwt_i = fwt[:H]
    fwt_s = fwt[H:]
    fb = fc_b.reshape(1, H)

    degp = _sc_deg(dsts)
    xs1 = _tc_prep(x_cat, wt1, degp)
    p1 = _sc_scatter(xs1.reshape(2 * N, H), srcs, dsts)
    xs2 = _tc_mid(p1, xs1, degp, b1, wt2)
    p2 = _sc_scatter(xs2.reshape(2 * N, H), srcs, dsts)
    return _tc_fin(p2, xs2, degp, b2, fwt_i, fwt_s, fb)


# speedup vs baseline: 4.0969x; 1.0547x over previous
import functools

import jax
import jax.numpy as jnp
from jax import lax
from jax.experimental import pallas as pl
from jax.experimental.pallas import tpu as pltpu
from jax.experimental.pallas import tpu_sc as plsc

N = 10000
E = 320000
H = 128

NC = 2
NS = 16
EP = 327680
EPT = EP // NS
K = 80
ST = 16
CPS = 16
NBUF = 4
DW = 8
NP = 10112
RPT = NP // NS

F32 = jnp.float32


def _mesh():
    return plsc.VectorSubcoreMesh(
        core_axis_name="c", subcore_axis_name="s",
        num_cores=NC, num_subcores=NS)


def _sc_deg(edg):
    @functools.partial(
        pl.kernel,
        out_type=jax.ShapeDtypeStruct((NC, NP, 16), F32),
        mesh=_mesh(),
        scratch_types=[
            pltpu.VMEM((2, CPS, K), jnp.int32),
            pltpu.VMEM((K, 16), F32),
            pltpu.VMEM_SHARED((NP, 16), F32),
            pltpu.SemaphoreType.DMA,
            pltpu.SemaphoreType.DMA,
        ],
    )
    def k(edg_hbm, out_hbm, slab, ones_v, acc, sem, lsem):
        c = lax.axis_index("c")
        s = lax.axis_index("s")
        base = s * RPT

        def _fill(val):
            def _f(i, carry):
                ones_v[i, :] = jnp.full((16,), val, F32)
                return carry
            lax.fori_loop(0, K, _f, 0)

        _fill(0.0)
        for j in range(RPT // K):
            pltpu.sync_copy(ones_v, acc.at[pl.ds(base + j * K, K)])
        pltpu.sync_copy(ones_v.at[pl.ds(0, RPT % K)],
                        acc.at[pl.ds(base + (RPT // K) * K, RPT % K)])
        plsc.subcore_barrier()
        _fill(1.0)

        def _fire(p, r):
            pltpu.async_copy(ones_v, acc.at[slab.at[p, r]], sem, add=True)

        def _drain(p, r):
            pltpu.make_async_copy(ones_v, acc.at[slab.at[p, r]], sem).wait()

        pltpu.sync_copy(edg_hbm.at[c, s, 0, 1], slab.at[0])
        for b in range(ST):
            p = b % 2
            if b + 1 < ST:
                pltpu.async_copy(edg_hbm.at[c, s, b + 1, 1],
                                 slab.at[1 - p], lsem)
            for q in range(DW):
                _fire(p, q)

            def _step(j, carry):
                _drain(p, j)
                _fire(p, j + DW)
                return carry
            lax.fori_loop(0, CPS - DW, _step, 0)
            for q in range(CPS - DW, CPS):
                _drain(p, q)
            if b + 1 < ST:
                pltpu.make_async_copy(edg_hbm.at[c, s, b + 1, 1],
                                      slab.at[1 - p], lsem).wait()
        plsc.subcore_barrier()

        pltpu.sync_copy(acc.at[pl.ds(base, RPT)],
                        out_hbm.at[c, pl.ds(base, RPT)])

    return k(edg)


def _sc_scatter(xs, edg):
    @functools.partial(
        pl.kernel,
        out_type=jax.ShapeDtypeStruct((NC, NP, H), F32),
        mesh=_mesh(),
        scratch_types=[
            pltpu.VMEM((2, 2, CPS, K), jnp.int32),
            pltpu.VMEM((NBUF, K, H), F32),
            pltpu.VMEM_SHARED((NP, H), F32),
            pltpu.SemaphoreType.DMA,
            pltpu.SemaphoreType.DMA,
            pltpu.SemaphoreType.DMA,
            pltpu.SemaphoreType.DMA,
            pltpu.SemaphoreType.DMA,
            pltpu.SemaphoreType.DMA,
            pltpu.SemaphoreType.DMA,
            pltpu.SemaphoreType.DMA,
            pltpu.SemaphoreType.DMA,
        ],
    )
    def k(xs_hbm, edg_hbm, out_hbm,
          slab, rows, acc,
          g0, g1, g2, g3, s0, s1, s2, s3, lsem):
        gs = (g0, g1, g2, g3)
        ss = (s0, s1, s2, s3)
        c = lax.axis_index("c")
        s = lax.axis_index("s")
        base = s * RPT

        def _gwait(q, p, r):
            pltpu.make_async_copy(
                xs_hbm.at[slab.at[p, 0, r]], rows.at[q], gs[q]).wait()

        def _swait(q, p, r):
            pltpu.make_async_copy(
                rows.at[q], acc.at[slab.at[p, 1, r]], ss[q]).wait()

        def _zrow(i, carry):
            for q in range(NBUF):
                for j in range(H // 16):
                    rows[q, i, pl.ds(j * 16, 16)] = jnp.zeros((16,), F32)
            return carry
        lax.fori_loop(0, K, _zrow, 0)
        for j in range(RPT // K):
            pltpu.sync_copy(rows.at[j % NBUF],
                            acc.at[pl.ds(base + j * K, K)])
        pltpu.sync_copy(rows.at[NBUF - 1, pl.ds(0, RPT % K)],
                        acc.at[pl.ds(base + (RPT // K) * K, RPT % K)])
        plsc.subcore_barrier()

        pltpu.sync_copy(edg_hbm.at[c, s, 0], slab.at[0])
        for q in range(NBUF):
            pltpu.async_copy(xs_hbm.at[slab.at[0, 0, q]], rows.at[q],
                             gs[q])
        for b in range(ST):
            p = b % 2
            if b + 1 < ST:
                pltpu.async_copy(edg_hbm.at[c, s, b + 1],
                                 slab.at[1 - p], lsem)

            def _round(j, carry):
                for q in range(NBUF):
                    r = j * NBUF + q
                    _gwait(q, p, r)
                    pltpu.async_copy(rows.at[q], acc.at[slab.at[p, 1, r]],
                                     ss[q], add=True)
                for q in range(NBUF):
                    r = j * NBUF + q
                    _swait(q, p, r)
                    pltpu.async_copy(xs_hbm.at[slab.at[p, 0, r + NBUF]],
                                     rows.at[q], gs[q])
                return carry
            lax.fori_loop(0, CPS // NBUF - 1, _round, 0)

            for q in range(NBUF):
                r = CPS - NBUF + q
                _gwait(q, p, r)
                pltpu.async_copy(rows.at[q], acc.at[slab.at[p, 1, r]],
                                 ss[q], add=True)
            if b + 1 < ST:
                pltpu.make_async_copy(edg_hbm.at[c, s, b + 1],
                                      slab.at[1 - p], lsem).wait()
            for q in range(NBUF):
                _swait(q, p, CPS - NBUF + q)
                if b + 1 < ST:
                    pltpu.async_copy(xs_hbm.at[slab.at[1 - p, 0, q]],
                                     rows.at[q], gs[q])
        plsc.subcore_barrier()

        pltpu.sync_copy(acc.at[pl.ds(base, RPT)],
                        out_hbm.at[c, pl.ds(base, RPT)])

    return k(xs, edg)


_R = 1000


def _g_row_spec():
    return pl.BlockSpec((1, _R, H), lambda g, i: (g, i, 0))


def _g_w_spec():
    return pl.BlockSpec((1, H, H), lambda g, i: (g, 0, 0))


def _g_deg_spec():
    return pl.BlockSpec((1, _R, 16), lambda g, i: (g, i, 0))


def _g_b_spec():
    return pl.BlockSpec((1, 1, H), lambda g, i: (g, 0, 0))


def _prep_body(x_ref, wt_ref, dg_ref, o_ref):
    dinv = lax.rsqrt(1.0 + dg_ref[0, :, 0])[:, None]
    o_ref[0] = dinv * jnp.dot(x_ref[0], wt_ref[0],
                              preferred_element_type=F32)


def _tc_prep(x, wt, degp):
    return pl.pallas_call(
        _prep_body,
        grid=(2, N // _R),
        in_specs=[_g_row_spec(), _g_w_spec(), _g_deg_spec()],
        out_specs=_g_row_spec(),
        out_shape=jax.ShapeDtypeStruct((2, N, H), F32),
    )(x, wt, degp)


def _mid_body(p_ref, xs_ref, dg_ref, b_ref, wt_ref, o_ref):
    dinv = lax.rsqrt(1.0 + dg_ref[0, :, 0])[:, None]
    h = jnp.maximum(dinv * (p_ref[0] + xs_ref[0]) + b_ref[0], 0.0)
    o_ref[0] = dinv * jnp.dot(h, wt_ref[0], preferred_element_type=F32)


def _tc_mid(parts, xs, degp, b, wt):
    return pl.pallas_call(
        _mid_body,
        grid=(2, N // _R),
        in_specs=[_g_row_spec(), _g_row_spec(), _g_deg_spec(),
                  _g_b_spec(), _g_w_spec()],
        out_specs=_g_row_spec(),
        out_shape=jax.ShapeDtypeStruct((2, N, H), F32),
    )(parts, xs, degp, b, wt)


def _fin_body(p_ref, xs_ref, dg_ref, b_ref, fwi_ref, fws_ref, fb_ref,
              o_ref):
    dinv_i = lax.rsqrt(1.0 + dg_ref[0, :, 0])[:, None]
    h_i = jnp.maximum(dinv_i * (p_ref[0] + xs_ref[0]) + b_ref[0], 0.0)
    dinv_s = lax.rsqrt(1.0 + dg_ref[1, :, 0])[:, None]
    h_s = jnp.maximum(dinv_s * (p_ref[1] + xs_ref[1]) + b_ref[1], 0.0)
    o_ref[...] = (jnp.dot(h_i, fwi_ref[...], preferred_element_type=F32)
                  + jnp.dot(h_s, fws_ref[...], preferred_element_type=F32)
                  + fb_ref[...])


def _tc_fin(parts, xs, degp, b, fwi, fws, fb):
    return pl.pallas_call(
        _fin_body,
        grid=(N // _R,),
        in_specs=[
            pl.BlockSpec((2, _R, H), lambda i: (0, i, 0)),
            pl.BlockSpec((2, _R, H), lambda i: (0, i, 0)),
            pl.BlockSpec((2, _R, 16), lambda i: (0, i, 0)),
            pl.BlockSpec((2, 1, H), lambda i: (0, 0, 0)),
            pl.BlockSpec((H, H), lambda i: (0, 0)),
            pl.BlockSpec((H, H), lambda i: (0, 0)),
            pl.BlockSpec((1, H), lambda i: (0, 0)),
        ],
        out_specs=pl.BlockSpec((_R, H), lambda i: (i, 0)),
        out_shape=jax.ShapeDtypeStruct((N, H), F32),
    )(parts, xs, degp, b, fwi, fws, fb)


def kernel(interaction_x, interaction_edge_index,
           similarity_x, similarity_edge_index,
           W_ic1, b_ic1, W_ic2, b_ic2,
           W_sc1, b_sc1, W_sc2, b_sc2, fc_W, fc_b):
    zpad = jnp.arange(EP - E, dtype=jnp.int32) % N
    npad = N + (jnp.arange(EP - E, dtype=jnp.int32) % (NP - N))
    shp = (NS, ST, CPS, K)

    def _edges(ei, off):
        s = (jnp.concatenate([ei[0], zpad]) + off).reshape(shp)
        d = jnp.concatenate([ei[1], npad]).reshape(shp)
        return jnp.stack([s, d], axis=2)

    edg = jnp.stack([_edges(interaction_edge_index, 0),
                     _edges(similarity_edge_index, N)])

    x_cat = jnp.stack([interaction_x, similarity_x])
    wt1 = jnp.stack([W_ic1.T, W_sc1.T])
    wt2 = jnp.stack([W_ic2.T, W_sc2.T])
    b1 = jnp.stack([b_ic1.reshape(1, H), b_sc1.reshape(1, H)])
    b2 = jnp.stack([b_ic2.reshape(1, H), b_sc2.reshape(1, H)])
    fwt = fc_W.T
    fwt_i = fwt[:H]
    fwt_s = fwt[H:]
    fb = fc_b.reshape(1, H)

    degp = _sc_deg(edg)
    xs1 = _tc_prep(x_cat, wt1, degp)
    p1 = _sc_scatter(xs1.reshape(2 * N, H), edg)
    xs2 = _tc_mid(p1, xs1, degp, b1, wt2)
    p2 = _sc_scatter(xs2.reshape(2 * N, H), edg)
    return _tc_fin(p2, xs2, degp, b2, fwt_i, fwt_s, fb)
